# baseline (speedup 1.0000x reference)
import jax
import jax.numpy as jnp
from jax import lax
from jax.experimental import pallas as pl
from jax.experimental.pallas import tpu as pltpu
from jax.experimental.pallas import tpu_sc as plsc

NUM_HEADS = 16
NREL = 66
TBL = NREL * NREL
NPIX = 1025 * 1025
SPAD = 1050640
BLK = 16 * 1025
NVEC = BLK // 16
TAIL_BASE = 64 * BLK
TAIL_N = NPIX - TAIL_BASE
TAIL_NVEC = 65
NC, NS, L = 2, 16, 16
HPC = NUM_HEADS // NC
COMB_WORDS = HPC * TBL + 16


def _sc_body(bh_hbm, bw_hbm, hidx_hbm, widx_hbm, out_hbm,
             bh_v, bw_v, comb, hbuf, wbuf, obuf0, obuf1,
             sem_in, sem_o0, sem_o1):
    cid = lax.axis_index("c")
    tid = lax.axis_index("s")

    pltpu.sync_copy(bh_hbm, bh_v)
    pltpu.sync_copy(bw_hbm, bw_v)
    hbase = cid * (HPC * 80)
    for hh in range(HPC):
        row = [bw_v[pl.ds(hbase + hh * 80 + v * 16, 16)] for v in range(5)]

        def build_a(a, _, hh=hh, row=row):
            s = plsc.load_gather(bh_v, [jnp.full((16,), hh * 80 + a, jnp.int32)
                                        + hbase])
            base = hh * TBL + a * NREL
            for v in range(5):
                comb[pl.ds(base + v * 16, 16)] = s + row[v]
            return 0

        lax.fori_loop(0, NREL, build_a, 0)

    obufs = (obuf0, obuf1)
    sems = (sem_o0, sem_o1)
    pending = [None, None]

    def do_block(base, n_dma, n_vec, n_dma_out=None):
        n_dma_out = n_dma if n_dma_out is None else n_dma_out
        cp1 = pltpu.async_copy(hidx_hbm.at[pl.ds(base, n_dma)],
                               hbuf.at[pl.ds(0, n_dma)], sem_in)
        cp2 = pltpu.async_copy(widx_hbm.at[pl.ds(base, n_dma)],
                               wbuf.at[pl.ds(0, n_dma)], sem_in)
        cp1.wait()
        cp2.wait()

        def cbody(v, _):
            off = v * 16
            c = hbuf[pl.ds(off, 16)] * NREL + wbuf[pl.ds(off, 16)]
            hbuf[pl.ds(off, 16)] = jnp.minimum(jnp.maximum(c, 0), TBL - 1)
            return 0

        lax.fori_loop(0, n_vec, cbody, 0)

        for hh in range(HPC):
            p = hh % 2
            ob = obufs[p]
            if pending[p] is not None:
                pending[p].wait()
                pending[p] = None

            def gbody(v, _, hh=hh, ob=ob):
                off = v * 16
                cv = hbuf[pl.ds(off, 16)]
                ob[pl.ds(off, 16)] = plsc.load_gather(comb, [cv + hh * TBL])
                return 0

            lax.fori_loop(0, n_vec, gbody, 0)
            pending[p] = pltpu.async_copy(
                ob.at[pl.ds(0, n_dma_out)],
                out_hbm.at[cid * HPC + hh, pl.ds(base, n_dma_out)], sems[p])

        for p in range(2):
            if pending[p] is not None:
                pending[p].wait()
                pending[p] = None

    for i in range(4):
        do_block((tid * 4 + i) * BLK, BLK, NVEC)

    @pl.when(tid == NS - 1)
    def _tail():
        do_block(TAIL_BASE, TAIL_N, TAIL_NVEC, n_dma_out=1032)


def kernel(bias_high, bias_width, h_index, w_index):
    bh_t = jnp.zeros((NUM_HEADS, 80), jnp.float32).at[:, :NREL].set(
        bias_high.T).reshape(NUM_HEADS * 80)
    bw_t = jnp.zeros((NUM_HEADS, 80), jnp.float32).at[:, :NREL].set(
        bias_width.T).reshape(NUM_HEADS * 80)
    h_flat = h_index.reshape(NPIX).astype(jnp.int32)
    w_flat = w_index.reshape(NPIX).astype(jnp.int32)

    run = pl.kernel(
        _sc_body,
        out_type=jax.ShapeDtypeStruct((NUM_HEADS, SPAD), jnp.float32),
        mesh=plsc.VectorSubcoreMesh(core_axis_name="c", subcore_axis_name="s",
                                    num_cores=NC, num_subcores=NS),
        compiler_params=pltpu.CompilerParams(use_tc_tiling_on_sc=False,
                                             needs_layout_passes=False),
        scratch_types=[
            pltpu.VMEM((NUM_HEADS * 80,), jnp.float32),
            pltpu.VMEM((NUM_HEADS * 80,), jnp.float32),
            pltpu.VMEM((COMB_WORDS,), jnp.float32),
            pltpu.VMEM((NVEC * 16,), jnp.int32),
            pltpu.VMEM((NVEC * 16,), jnp.int32),
            pltpu.VMEM((NVEC * 16,), jnp.float32),
            pltpu.VMEM((NVEC * 16,), jnp.float32),
            pltpu.SemaphoreType.DMA,
            pltpu.SemaphoreType.DMA,
            pltpu.SemaphoreType.DMA,
        ],
    )
    out = run(bh_t, bw_t, h_flat, w_flat)
    return out[:, :NPIX].reshape(NUM_HEADS, 1025, 1025)


# speedup vs baseline: 1.0001x; 1.0001x over previous
import jax
import jax.numpy as jnp
from jax import lax
from jax.experimental import pallas as pl
from jax.experimental.pallas import tpu as pltpu
from jax.experimental.pallas import tpu_sc as plsc

NUM_HEADS = 16
NREL = 66
TBL = NREL * NREL
NPIX = 1025 * 1025
SPAD = 1050640
BLK = 16 * 1025
NVEC = BLK // 16
TAIL_BASE = 64 * BLK
TAIL_N = NPIX - TAIL_BASE
TAIL_NVEC = 65
NC, NS, L = 2, 16, 16
HPC = NUM_HEADS // NC
COMB_WORDS = HPC * TBL + 16


def _sc_body(bh_hbm, bw_hbm, hidx_hbm, widx_hbm, out_hbm,
             bh_v, bw_v, comb, hbuf, wbuf, obuf0, obuf1,
             sem_in, sem_o0, sem_o1):
    cid = lax.axis_index("c")
    tid = lax.axis_index("s")

    pltpu.sync_copy(bh_hbm, bh_v)
    pltpu.sync_copy(bw_hbm, bw_v)
    hbase = cid * (HPC * 80)
    for hh in range(HPC):
        row = [bw_v[pl.ds(hbase + hh * 80 + v * 16, 16)] for v in range(5)]

        def build_a(a, _, hh=hh, row=row):
            s = plsc.load_gather(bh_v, [jnp.full((16,), hh * 80 + a, jnp.int32)
                                        + hbase])
            base = hh * TBL + a * NREL
            for v in range(5):
                comb[pl.ds(base + v * 16, 16)] = s + row[v]
            return 0

        lax.fori_loop(0, NREL, build_a, 0)

    obufs = (obuf0, obuf1)
    sems = (sem_o0, sem_o1)
    pending = [None, None]

    def do_block(base, n_dma, n_vec, n_dma_out=None):
        n_dma_out = n_dma if n_dma_out is None else n_dma_out
        cp1 = pltpu.async_copy(hidx_hbm.at[pl.ds(base, n_dma)],
                               hbuf.at[pl.ds(0, n_dma)], sem_in)
        cp2 = pltpu.async_copy(widx_hbm.at[pl.ds(base, n_dma)],
                               wbuf.at[pl.ds(0, n_dma)], sem_in)
        cp1.wait()
        cp2.wait()

        def cbody(v, _):
            off = v * 16
            c = hbuf[pl.ds(off, 16)] * NREL + wbuf[pl.ds(off, 16)]
            hbuf[pl.ds(off, 16)] = jnp.minimum(jnp.maximum(c, 0), TBL - 1)
            return 0

        lax.fori_loop(0, n_vec, cbody, 0)

        for hh in range(HPC):
            p = hh % 2
            ob = obufs[p]
            if pending[p] is not None:
                pending[p].wait()
                pending[p] = None

            def gbody(v, _, hh=hh, ob=ob):
                off = v * 16
                cv = hbuf[pl.ds(off, 16)]
                ob[pl.ds(off, 16)] = plsc.load_gather(comb, [cv + hh * TBL])
                return 0

            lax.fori_loop(0, n_vec, gbody, 0)
            pending[p] = pltpu.async_copy(
                ob.at[pl.ds(0, n_dma_out)],
                out_hbm.at[pl.ds((cid * HPC + hh) * SPAD + base, n_dma_out)],
                sems[p])

        for p in range(2):
            if pending[p] is not None:
                pending[p].wait()
                pending[p] = None

    for i in range(4):
        do_block((tid * 4 + i) * BLK, BLK, NVEC)

    @pl.when(tid == NS - 1)
    def _tail():
        do_block(TAIL_BASE, TAIL_N, TAIL_NVEC, n_dma_out=1032)


def kernel(bias_high, bias_width, h_index, w_index):
    bh_t = jnp.zeros((NUM_HEADS, 80), jnp.float32).at[:, :NREL].set(
        bias_high.T).reshape(NUM_HEADS * 80)
    bw_t = jnp.zeros((NUM_HEADS, 80), jnp.float32).at[:, :NREL].set(
        bias_width.T).reshape(NUM_HEADS * 80)
    h_flat = h_index.reshape(NPIX).astype(jnp.int32)
    w_flat = w_index.reshape(NPIX).astype(jnp.int32)

    run = pl.kernel(
        _sc_body,
        out_type=jax.ShapeDtypeStruct((NUM_HEADS * SPAD,), jnp.float32),
        mesh=plsc.VectorSubcoreMesh(core_axis_name="c", subcore_axis_name="s",
                                    num_cores=NC, num_subcores=NS),
        compiler_params=pltpu.CompilerParams(use_tc_tiling_on_sc=False,
                                             needs_layout_passes=False),
        scratch_types=[
            pltpu.VMEM((NUM_HEADS * 80,), jnp.float32),
            pltpu.VMEM((NUM_HEADS * 80,), jnp.float32),
            pltpu.VMEM((COMB_WORDS,), jnp.float32),
            pltpu.VMEM((NVEC * 16,), jnp.int32),
            pltpu.VMEM((NVEC * 16,), jnp.int32),
            pltpu.VMEM((NVEC * 16,), jnp.float32),
            pltpu.VMEM((NVEC * 16,), jnp.float32),
            pltpu.SemaphoreType.DMA,
            pltpu.SemaphoreType.DMA,
            pltpu.SemaphoreType.DMA,
        ],
    )
    out = run(bh_t, bw_t, h_flat, w_flat)
    return out.reshape(NUM_HEADS, SPAD)[:, :NPIX].reshape(NUM_HEADS, 1025, 1025)


# speedup vs baseline: 12.3039x; 12.3027x over previous
import jax
import jax.numpy as jnp
from jax import lax
from jax.experimental import pallas as pl
from jax.experimental.pallas import tpu as pltpu
from jax.experimental.pallas import tpu_sc as plsc

NUM_HEADS = 16
NREL = 66
TBL = NREL * NREL
N = 1025
RPT = 8
NRT = 128
VPR = 72
BW = VPR * 16
NC, NS, L = 2, 16, 16
HPC = NUM_HEADS // NC
COMB_WORDS = HPC * TBL + 16


def _sc_body(bh_hbm, bw_hbm, hidx_hbm, widx_hbm, out_hbm,
             bh_v, bw_v, comb, hbuf, wbuf, obuf0, obuf1,
             sem_in, sem_o0, sem_o1):
    cid = lax.axis_index("c")
    tid = lax.axis_index("s")

    pltpu.sync_copy(bh_hbm, bh_v)
    pltpu.sync_copy(bw_hbm, bw_v)
    hbase = cid * (HPC * 80)
    for hh in range(HPC):
        row = [bw_v[pl.ds(hbase + hh * 80 + v * 16, 16)] for v in range(5)]

        def build_a(a, _, hh=hh, row=row):
            s = plsc.load_gather(bh_v, [jnp.full((16,), hh * 80 + a, jnp.int32)
                                        + hbase])
            base = hh * TBL + a * NREL
            for v in range(5):
                comb[pl.ds(base + v * 16, 16)] = s + row[v]
            return 0

        lax.fori_loop(0, NREL, build_a, 0)

    obufs = (obuf0, obuf1)
    sems = (sem_o0, sem_o1)

    def do_tile(r0, nrows):
        cp1 = pltpu.async_copy(hidx_hbm.at[pl.ds(r0, nrows), pl.ds(0, BW)],
                               hbuf.at[pl.ds(0, nrows), :], sem_in)
        cp2 = pltpu.async_copy(widx_hbm.at[pl.ds(r0, nrows), pl.ds(0, BW)],
                               wbuf.at[pl.ds(0, nrows), :], sem_in)
        cp1.wait()
        cp2.wait()

        def cbody(v, _):
            off = v * 16
            for r in range(nrows):
                c = hbuf[r, pl.ds(off, 16)] * NREL + wbuf[r, pl.ds(off, 16)]
                hbuf[r, pl.ds(off, 16)] = jnp.minimum(jnp.maximum(c, 0),
                                                      TBL - 1)
            return 0

        lax.fori_loop(0, VPR, cbody, 0)

        pending = [None, None]
        for hh in range(HPC):
            p = hh % 2
            ob = obufs[p]
            if pending[p] is not None:
                pending[p].wait()
                pending[p] = None

            def gbody(v, _, hh=hh, ob=ob):
                off = v * 16
                for r in range(nrows):
                    cv = hbuf[r, pl.ds(off, 16)]
                    ob[r, pl.ds(off, 16)] = plsc.load_gather(comb,
                                                             [cv + hh * TBL])
                return 0

            lax.fori_loop(0, VPR, gbody, 0)
            pending[p] = pltpu.async_copy(
                ob.at[pl.ds(0, nrows), :],
                out_hbm.at[cid * HPC + hh, pl.ds(r0, nrows), pl.ds(0, BW)],
                sems[p])

        for p in range(2):
            if pending[p] is not None:
                pending[p].wait()

    def tile_body(i, _):
        do_tile((tid * 8 + i) * RPT, RPT)
        return 0

    lax.fori_loop(0, 8, tile_body, 0)

    @pl.when(tid == NS - 1)
    def _tail():
        do_tile(NRT * RPT, 1)


def kernel(bias_high, bias_width, h_index, w_index):
    bh_t = jnp.zeros((NUM_HEADS, 80), jnp.float32).at[:, :NREL].set(
        bias_high.T).reshape(NUM_HEADS * 80)
    bw_t = jnp.zeros((NUM_HEADS, 80), jnp.float32).at[:, :NREL].set(
        bias_width.T).reshape(NUM_HEADS * 80)

    run = pl.kernel(
        _sc_body,
        out_type=jax.ShapeDtypeStruct((NUM_HEADS, N, N), jnp.float32),
        mesh=plsc.VectorSubcoreMesh(core_axis_name="c", subcore_axis_name="s",
                                    num_cores=NC, num_subcores=NS),
        compiler_params=pltpu.CompilerParams(use_tc_tiling_on_sc=True,
                                             needs_layout_passes=False),
        scratch_types=[
            pltpu.VMEM((NUM_HEADS * 80,), jnp.float32),
            pltpu.VMEM((NUM_HEADS * 80,), jnp.float32),
            pltpu.VMEM((COMB_WORDS,), jnp.float32),
            pltpu.VMEM((RPT, BW), jnp.int32),
            pltpu.VMEM((RPT, BW), jnp.int32),
            pltpu.VMEM((RPT, BW), jnp.float32),
            pltpu.VMEM((RPT, BW), jnp.float32),
            pltpu.SemaphoreType.DMA,
            pltpu.SemaphoreType.DMA,
            pltpu.SemaphoreType.DMA,
        ],
    )
    return run(bh_t, bw_t, h_index.astype(jnp.int32), w_index.astype(jnp.int32))


# speedup vs baseline: 15.0307x; 1.2216x over previous
import jax
import jax.numpy as jnp
from jax import lax
from jax.experimental import pallas as pl
from jax.experimental.pallas import tpu as pltpu
from jax.experimental.pallas import tpu_sc as plsc

NUM_HEADS = 16
NREL = 66
TBL = NREL * NREL
TBLP = TBL + 4
N = 1025
RPT = 8
NRT = 128
VPR = 72
BW = VPR * 16
NC, NS, L = 2, 16, 16
HPC = NUM_HEADS // NC
COMB_WORDS = HPC * TBLP + 16


def _sc_body(bh_hbm, bw_hbm, hidx_hbm, widx_hbm, out_hbm,
             bh_v, bw_v, comb, hbuf, wbuf, obuf,
             sem_in, sem_out):
    cid = lax.axis_index("c")
    tid = lax.axis_index("s")

    pltpu.sync_copy(bh_hbm, bh_v)
    pltpu.sync_copy(bw_hbm, bw_v)
    hbase = cid * (HPC * 80)
    for hh in range(HPC):
        row = [bw_v[pl.ds(hbase + hh * 80 + v * 16, 16)] for v in range(5)]

        def build_a(a, _, hh=hh, row=row):
            s = plsc.load_gather(bh_v, [jnp.full((16,), hh * 80 + a, jnp.int32)
                                        + hbase])
            base = hh * TBLP + a * NREL
            for v in range(5):
                comb[pl.ds(base + v * 16, 16)] = s + row[v]
            return 0

        lax.fori_loop(0, NREL, build_a, 0)

    def do_tile(r0, nrows, first, prev_heads):
        cp1 = pltpu.async_copy(hidx_hbm.at[pl.ds(r0, nrows), pl.ds(0, BW)],
                               hbuf.at[pl.ds(0, nrows), :], sem_in)
        cp2 = pltpu.async_copy(widx_hbm.at[pl.ds(r0, nrows), pl.ds(0, BW)],
                               wbuf.at[pl.ds(0, nrows), :], sem_in)
        cp1.wait()
        cp2.wait()

        def cbody(v, _):
            off = v * 16
            for r in range(nrows):
                c = hbuf[r, pl.ds(off, 16)] * NREL + wbuf[r, pl.ds(off, 16)]
                hbuf[r, pl.ds(off, 16)] = jnp.minimum(jnp.maximum(c, 0),
                                                      TBL - 1)
            return 0

        lax.fori_loop(0, VPR, cbody, 0)

        @pl.when(jnp.logical_not(first))
        def _drain():
            for _ in range(prev_heads):
                pltpu.make_async_copy(
                    out_hbm.at[0, pl.ds(0, RPT), pl.ds(0, BW)],
                    obuf.at[0], sem_out).wait()

        def gbody(v, _):
            off = v * 16
            for r in range(nrows):
                cv = hbuf[r, pl.ds(off, 16)]
                for hh in range(HPC):
                    obuf[hh, r, pl.ds(off, 16)] = plsc.load_gather(
                        comb.at[pl.ds(hh * TBLP, TBL)], [cv])
            return 0

        lax.fori_loop(0, VPR, gbody, 0)

        for hh in range(HPC):
            pltpu.async_copy(
                obuf.at[hh, pl.ds(0, nrows), :],
                out_hbm.at[cid * HPC + hh, pl.ds(r0, nrows), pl.ds(0, BW)],
                sem_out)

    def tile_body(i, _):
        do_tile((tid * 8 + i) * RPT, RPT, i == 0, HPC)
        return 0

    lax.fori_loop(0, 8, tile_body, 0)

    for _ in range(HPC):
        pltpu.make_async_copy(out_hbm.at[0, pl.ds(0, RPT), pl.ds(0, BW)],
                              obuf.at[0], sem_out).wait()

    @pl.when(tid == NS - 1)
    def _tail():
        do_tile(NRT * RPT, 1, True, 0)
        for _ in range(HPC):
            pltpu.make_async_copy(out_hbm.at[0, pl.ds(0, 1), pl.ds(0, BW)],
                                  obuf.at[0, pl.ds(0, 1), :], sem_out).wait()


def kernel(bias_high, bias_width, h_index, w_index):
    bh_t = jnp.zeros((NUM_HEADS, 80), jnp.float32).at[:, :NREL].set(
        bias_high.T).reshape(NUM_HEADS * 80)
    bw_t = jnp.zeros((NUM_HEADS, 80), jnp.float32).at[:, :NREL].set(
        bias_width.T).reshape(NUM_HEADS * 80)

    run = pl.kernel(
        _sc_body,
        out_type=jax.ShapeDtypeStruct((NUM_HEADS, N, N), jnp.float32),
        mesh=plsc.VectorSubcoreMesh(core_axis_name="c", subcore_axis_name="s",
                                    num_cores=NC, num_subcores=NS),
        compiler_params=pltpu.CompilerParams(use_tc_tiling_on_sc=True,
                                             needs_layout_passes=False),
        scratch_types=[
            pltpu.VMEM((NUM_HEADS * 80,), jnp.float32),
            pltpu.VMEM((NUM_HEADS * 80,), jnp.float32),
            pltpu.VMEM((COMB_WORDS,), jnp.float32),
            pltpu.VMEM((RPT, BW), jnp.int32),
            pltpu.VMEM((RPT, BW), jnp.int32),
            pltpu.VMEM((HPC, RPT, BW), jnp.float32),
            pltpu.SemaphoreType.DMA,
            pltpu.SemaphoreType.DMA,
        ],
    )
    return run(bh_t, bw_t, h_index.astype(jnp.int32), w_index.astype(jnp.int32))


# speedup vs baseline: 18.6330x; 1.2397x over previous
import jax
import jax.numpy as jnp
from jax import lax
from jax.experimental import pallas as pl
from jax.experimental.pallas import tpu as pltpu
from jax.experimental.pallas import tpu_sc as plsc

NUM_HEADS = 16
NREL = 66
TBL = NREL * NREL
TBLP = TBL + 4
N = 1025
RPT = 8
NRT = 128
VPR = 72
BW = VPR * 16
NC, NS, L = 2, 16, 16
HPC = NUM_HEADS // NC
COMB_WORDS = HPC * TBLP + 16


def _sc_body(bh_hbm, bw_hbm, hidx_hbm, widx_hbm, out_hbm,
             bh_v, bw_v, comb, hbuf, wbuf, obuf,
             sem_in, sem_out):
    cid = lax.axis_index("c")
    tid = lax.axis_index("s")

    pltpu.sync_copy(bh_hbm, bh_v)
    pltpu.sync_copy(bw_hbm, bw_v)
    hbase = cid * (HPC * 80)
    for hh in range(HPC):
        row = [bw_v[pl.ds(hbase + hh * 80 + v * 16, 16)] for v in range(5)]

        def build_a(a, _, hh=hh, row=row):
            s = plsc.load_gather(bh_v, [jnp.full((16,), hh * 80 + a, jnp.int32)
                                        + hbase])
            base = hh * TBLP + a * NREL
            for v in range(5):
                comb[pl.ds(base + v * 16, 16)] = s + row[v]
            return 0

        lax.fori_loop(0, NREL, build_a, 0)

    def do_tile(r0, nrows, first, prev_heads):
        cp1 = pltpu.async_copy(hidx_hbm.at[pl.ds(r0, nrows), pl.ds(0, BW)],
                               hbuf.at[pl.ds(0, nrows), :], sem_in)
        cp2 = pltpu.async_copy(widx_hbm.at[pl.ds(r0, nrows), pl.ds(0, BW)],
                               wbuf.at[pl.ds(0, nrows), :], sem_in)
        cp1.wait()
        cp2.wait()

        def cbody(v, _):
            off = v * 16
            for r in range(nrows):
                c = hbuf[r, pl.ds(off, 16)] * NREL + wbuf[r, pl.ds(off, 16)]
                hbuf[r, pl.ds(off, 16)] = jnp.minimum(jnp.maximum(c, 0),
                                                      TBL - 1)
            return 0

        lax.fori_loop(0, VPR, cbody, 0)

        @pl.when(jnp.logical_not(first))
        def _drain():
            pltpu.make_async_copy(
                out_hbm.at[pl.ds(0, RPT), pl.ds(0, HPC), pl.ds(0, BW)],
                obuf, sem_out).wait()

        def gbody(v, _):
            off = v * 16
            for r in range(nrows):
                cv = hbuf[r, pl.ds(off, 16)]
                for hh in range(HPC):
                    obuf[r, hh, pl.ds(off, 16)] = plsc.load_gather(
                        comb.at[pl.ds(hh * TBLP, TBL)], [cv])
            return 0

        lax.fori_loop(0, VPR, gbody, 0)

        pltpu.async_copy(
            obuf.at[pl.ds(0, nrows), :, :],
            out_hbm.at[pl.ds(r0, nrows), pl.ds(cid * HPC, HPC), pl.ds(0, BW)],
            sem_out)

    def tile_body(i, _):
        do_tile((tid * 8 + i) * RPT, RPT, i == 0, HPC)
        return 0

    lax.fori_loop(0, 8, tile_body, 0)

    pltpu.make_async_copy(out_hbm.at[pl.ds(0, RPT), pl.ds(0, HPC), pl.ds(0, BW)],
                          obuf, sem_out).wait()

    @pl.when(tid == NS - 1)
    def _tail():
        do_tile(NRT * RPT, 1, True, 0)
        pltpu.make_async_copy(
            out_hbm.at[pl.ds(0, 1), pl.ds(0, HPC), pl.ds(0, BW)],
            obuf.at[pl.ds(0, 1), :, :], sem_out).wait()


def kernel(bias_high, bias_width, h_index, w_index):
    bh_t = jnp.zeros((NUM_HEADS, 80), jnp.float32).at[:, :NREL].set(
        bias_high.T).reshape(NUM_HEADS * 80)
    bw_t = jnp.zeros((NUM_HEADS, 80), jnp.float32).at[:, :NREL].set(
        bias_width.T).reshape(NUM_HEADS * 80)

    run = pl.kernel(
        _sc_body,
        out_type=jax.ShapeDtypeStruct((N, NUM_HEADS, N), jnp.float32),
        mesh=plsc.VectorSubcoreMesh(core_axis_name="c", subcore_axis_name="s",
                                    num_cores=NC, num_subcores=NS),
        compiler_params=pltpu.CompilerParams(use_tc_tiling_on_sc=True,
                                             needs_layout_passes=False),
        scratch_types=[
            pltpu.VMEM((NUM_HEADS * 80,), jnp.float32),
            pltpu.VMEM((NUM_HEADS * 80,), jnp.float32),
            pltpu.VMEM((COMB_WORDS,), jnp.float32),
            pltpu.VMEM((RPT, BW), jnp.int32),
            pltpu.VMEM((RPT, BW), jnp.int32),
            pltpu.VMEM((RPT, HPC, BW), jnp.float32),
            pltpu.SemaphoreType.DMA,
            pltpu.SemaphoreType.DMA,
        ],
    )
    out_ihj = run(bh_t, bw_t, h_index.astype(jnp.int32),
                  w_index.astype(jnp.int32))
    return jnp.transpose(out_ihj, (1, 0, 2))


# speedup vs baseline: 35.0716x; 1.8822x over previous
import jax
import jax.numpy as jnp
from jax import lax
from jax.experimental import pallas as pl
from jax.experimental.pallas import tpu as pltpu
from jax.experimental.pallas import tpu_sc as plsc

NUM_HEADS = 16
NREL = 66
TBL = NREL * NREL
TBLP = TBL + 4
N = 1025
RPT = 8
NRT = 128
VPR = 72
BW = VPR * 16
NC, NS, L = 2, 16, 16
HPC = NUM_HEADS // NC
COMB_WORDS = HPC * TBLP + 16


def _sc_body(bh_hbm, bw_hbm, hidx_hbm, widx_hbm, out_hbm,
             bh_v, bw_v, comb, hbuf, wbuf, obuf,
             sem_in, sem_out):
    cid = lax.axis_index("c")
    tid = lax.axis_index("s")

    pltpu.sync_copy(bh_hbm, bh_v)
    pltpu.sync_copy(bw_hbm, bw_v)
    hbase = cid * (HPC * 80)
    for hh in range(HPC):
        row = [bw_v[pl.ds(hbase + hh * 80 + v * 16, 16)] for v in range(5)]

        def build_a(a, _, hh=hh, row=row):
            s = plsc.load_gather(bh_v, [jnp.full((16,), hh * 80 + a, jnp.int32)
                                        + hbase])
            base = hh * TBLP + a * NREL
            for v in range(5):
                comb[pl.ds(base + v * 16, 16)] = s + row[v]
            return 0

        lax.fori_loop(0, NREL, build_a, 0)

    def do_tile(r0, nrows, first, prev_heads):
        cp1 = pltpu.async_copy(hidx_hbm.at[pl.ds(r0, nrows), pl.ds(0, BW)],
                               hbuf.at[pl.ds(0, nrows), :], sem_in)
        cp2 = pltpu.async_copy(widx_hbm.at[pl.ds(r0, nrows), pl.ds(0, BW)],
                               wbuf.at[pl.ds(0, nrows), :], sem_in)
        cp1.wait()
        cp2.wait()

        def cbody(v, _):
            off = v * 16
            for r in range(nrows):
                c = hbuf[r, pl.ds(off, 16)] * NREL + wbuf[r, pl.ds(off, 16)]
                hbuf[r, pl.ds(off, 16)] = jnp.minimum(jnp.maximum(c, 0),
                                                      TBL - 1)
            return 0

        lax.fori_loop(0, VPR, cbody, 0)

        @pl.when(jnp.logical_not(first))
        def _drain():
            pltpu.make_async_copy(
                out_hbm.at[pl.ds(0, RPT), pl.ds(0, HPC), pl.ds(0, BW)],
                obuf, sem_out).wait()

        def gbody(v, _):
            off = v * 16
            for r in range(nrows):
                cv = hbuf[r, pl.ds(off, 16)]
                gs = [plsc.load_gather(comb.at[pl.ds(hh * TBLP, TBL)], [cv])
                      for hh in range(HPC)]
                for hh in range(HPC):
                    obuf[r, hh, pl.ds(off, 16)] = gs[hh]
            return 0

        lax.fori_loop(0, VPR, gbody, 0)

        pltpu.async_copy(
            obuf.at[pl.ds(0, nrows), :, :],
            out_hbm.at[pl.ds(r0, nrows), pl.ds(cid * HPC, HPC), pl.ds(0, BW)],
            sem_out)

    def tile_body(i, _):
        do_tile((tid * 8 + i) * RPT, RPT, i == 0, HPC)
        return 0

    lax.fori_loop(0, 8, tile_body, 0)

    pltpu.make_async_copy(out_hbm.at[pl.ds(0, RPT), pl.ds(0, HPC), pl.ds(0, BW)],
                          obuf, sem_out).wait()

    @pl.when(tid == NS - 1)
    def _tail():
        do_tile(NRT * RPT, 1, True, 0)
        pltpu.make_async_copy(
            out_hbm.at[pl.ds(0, 1), pl.ds(0, HPC), pl.ds(0, BW)],
            obuf.at[pl.ds(0, 1), :, :], sem_out).wait()


def kernel(bias_high, bias_width, h_index, w_index):
    bh_t = jnp.zeros((NUM_HEADS, 80), jnp.float32).at[:, :NREL].set(
        bias_high.T).reshape(NUM_HEADS * 80)
    bw_t = jnp.zeros((NUM_HEADS, 80), jnp.float32).at[:, :NREL].set(
        bias_width.T).reshape(NUM_HEADS * 80)

    run = pl.kernel(
        _sc_body,
        out_type=jax.ShapeDtypeStruct((N, NUM_HEADS, N), jnp.float32),
        mesh=plsc.VectorSubcoreMesh(core_axis_name="c", subcore_axis_name="s",
                                    num_cores=NC, num_subcores=NS),
        compiler_params=pltpu.CompilerParams(use_tc_tiling_on_sc=True,
                                             needs_layout_passes=False),
        scratch_types=[
            pltpu.VMEM((NUM_HEADS * 80,), jnp.float32),
            pltpu.VMEM((NUM_HEADS * 80,), jnp.float32),
            pltpu.VMEM((COMB_WORDS,), jnp.float32),
            pltpu.VMEM((RPT, BW), jnp.int32),
            pltpu.VMEM((RPT, BW), jnp.int32),
            pltpu.VMEM((RPT, HPC, BW), jnp.float32),
            pltpu.SemaphoreType.DMA,
            pltpu.SemaphoreType.DMA,
        ],
    )
    out_ihj = run(bh_t, bw_t, h_index.astype(jnp.int32),
                  w_index.astype(jnp.int32))
    return jnp.transpose(out_ihj, (1, 0, 2))


# speedup vs baseline: 43.2042x; 1.2319x over previous
import jax
import jax.numpy as jnp
from jax import lax
from jax.experimental import pallas as pl
from jax.experimental.pallas import tpu as pltpu
from jax.experimental.pallas import tpu_sc as plsc

NUM_HEADS = 16
NREL = 66
TBL = NREL * NREL
TBLP = TBL + 4
N = 1025
RPT = 8
NRT = 128
VPR = 72
BW = VPR * 16
NC, NS, L = 2, 16, 16
HPC = NUM_HEADS // NC
COMB_WORDS = HPC * TBLP + 16


def _sc_body(bh_hbm, bw_hbm, hidx_hbm, widx_hbm, out_hbm,
             bh_v, bw_v, comb, hbuf, wbuf, obuf,
             sem_in, sem_out):
    cid = lax.axis_index("c")
    tid = lax.axis_index("s")

    pltpu.sync_copy(bh_hbm, bh_v)
    pltpu.sync_copy(bw_hbm, bw_v)
    hbase = cid * (HPC * 80)
    for hh in range(HPC):
        row = [bw_v[pl.ds(hbase + hh * 80 + v * 16, 16)] for v in range(5)]

        def build_a(a, _, hh=hh, row=row):
            s = plsc.load_gather(bh_v, [jnp.full((16,), hh * 80 + a, jnp.int32)
                                        + hbase])
            base = hh * TBLP + a * NREL
            for v in range(5):
                comb[pl.ds(base + v * 16, 16)] = s + row[v]
            return 0

        lax.fori_loop(0, NREL, build_a, 0)

    def do_tile(r0, nrows, first, prev_heads):
        cp1 = pltpu.async_copy(hidx_hbm.at[pl.ds(r0, nrows), pl.ds(0, BW)],
                               hbuf.at[pl.ds(0, nrows), :], sem_in)
        cp2 = pltpu.async_copy(widx_hbm.at[pl.ds(r0, nrows), pl.ds(0, BW)],
                               wbuf.at[pl.ds(0, nrows), :], sem_in)
        cp1.wait()
        cp2.wait()

        @pl.when(jnp.logical_not(first))
        def _drain():
            pltpu.make_async_copy(
                out_hbm.at[pl.ds(0, RPT), pl.ds(0, HPC), pl.ds(0, BW)],
                obuf, sem_out).wait()

        def gbody(v, _):
            off = v * 16
            cvs = []
            for r in range(nrows):
                c = hbuf[r, pl.ds(off, 16)] * NREL + wbuf[r, pl.ds(off, 16)]
                cvs.append(jnp.minimum(jnp.maximum(c, 0), TBL - 1))
            pgs = [plsc.load_gather(comb.at[pl.ds(hh * TBLP, TBL)], [cvs[0]])
                   for hh in range(HPC)]
            for r in range(1, nrows):
                ngs = []
                for hh in range(HPC):
                    g = plsc.load_gather(comb.at[pl.ds(hh * TBLP, TBL)],
                                         [cvs[r]])
                    obuf[r - 1, hh, pl.ds(off, 16)] = pgs[hh]
                    ngs.append(g)
                pgs = ngs
            for hh in range(HPC):
                obuf[nrows - 1, hh, pl.ds(off, 16)] = pgs[hh]
            return 0

        lax.fori_loop(0, VPR // 2, lambda u, c: (gbody(2 * u, c),
                                                  gbody(2 * u + 1, c))[1], 0)

        pltpu.async_copy(
            obuf.at[pl.ds(0, nrows), :, :],
            out_hbm.at[pl.ds(r0, nrows), pl.ds(cid * HPC, HPC), pl.ds(0, BW)],
            sem_out)

    def tile_body(i, _):
        do_tile((tid * 8 + i) * RPT, RPT, i == 0, HPC)
        return 0

    lax.fori_loop(0, 8, tile_body, 0)

    pltpu.make_async_copy(out_hbm.at[pl.ds(0, RPT), pl.ds(0, HPC), pl.ds(0, BW)],
                          obuf, sem_out).wait()

    @pl.when(tid == NS - 1)
    def _tail():
        do_tile(NRT * RPT, 1, True, 0)
        pltpu.make_async_copy(
            out_hbm.at[pl.ds(0, 1), pl.ds(0, HPC), pl.ds(0, BW)],
            obuf.at[pl.ds(0, 1), :, :], sem_out).wait()


def kernel(bias_high, bias_width, h_index, w_index):
    bh_t = jnp.zeros((NUM_HEADS, 80), jnp.float32).at[:, :NREL].set(
        bias_high.T).reshape(NUM_HEADS * 80)
    bw_t = jnp.zeros((NUM_HEADS, 80), jnp.float32).at[:, :NREL].set(
        bias_width.T).reshape(NUM_HEADS * 80)

    run = pl.kernel(
        _sc_body,
        out_type=jax.ShapeDtypeStruct((N, NUM_HEADS, N), jnp.float32),
        mesh=plsc.VectorSubcoreMesh(core_axis_name="c", subcore_axis_name="s",
                                    num_cores=NC, num_subcores=NS),
        compiler_params=pltpu.CompilerParams(use_tc_tiling_on_sc=True,
                                             needs_layout_passes=False),
        scratch_types=[
            pltpu.VMEM((NUM_HEADS * 80,), jnp.float32),
            pltpu.VMEM((NUM_HEADS * 80,), jnp.float32),
            pltpu.VMEM((COMB_WORDS,), jnp.float32),
            pltpu.VMEM((RPT, BW), jnp.int32),
            pltpu.VMEM((RPT, BW), jnp.int32),
            pltpu.VMEM((RPT, HPC, BW), jnp.float32),
            pltpu.SemaphoreType.DMA,
            pltpu.SemaphoreType.DMA,
        ],
    )
    out_ihj = run(bh_t, bw_t, h_index.astype(jnp.int32),
                  w_index.astype(jnp.int32))
    return jnp.transpose(out_ihj, (1, 0, 2))


# speedup vs baseline: 43.9510x; 1.0173x over previous
import jax
import jax.numpy as jnp
from jax import lax
from jax.experimental import pallas as pl
from jax.experimental.pallas import tpu as pltpu
from jax.experimental.pallas import tpu_sc as plsc

NUM_HEADS = 16
NREL = 66
TBL = NREL * NREL
TBLP = TBL + 4
N = 1025
RPT = 8
NRT = 128
VPR = 72
BW = VPR * 16
NC, NS, L = 2, 16, 16
HPC = NUM_HEADS // NC
COMB_WORDS = HPC * TBLP + 16


def _sc_body(bh_hbm, bw_hbm, hidx_hbm, widx_hbm, out_hbm,
             bh_v, bw_v, comb, hbuf, wbuf, obuf,
             sem_in, sem_out):
    cid = lax.axis_index("c")
    tid = lax.axis_index("s")

    pltpu.sync_copy(bh_hbm, bh_v)
    pltpu.sync_copy(bw_hbm, bw_v)
    hbase = cid * (HPC * 80)
    for hh in range(HPC):
        row = [bw_v[pl.ds(hbase + hh * 80 + v * 16, 16)] for v in range(5)]

        def build_a(a, _, hh=hh, row=row):
            s = plsc.load_gather(bh_v, [jnp.full((16,), hh * 80 + a, jnp.int32)
                                        + hbase])
            base = hh * TBLP + a * NREL
            for v in range(5):
                comb[pl.ds(base + v * 16, 16)] = s + row[v]
            return 0

        lax.fori_loop(0, NREL, build_a, 0)

    def do_tile(r0, nrows, first, prev_heads):
        cp1 = pltpu.async_copy(hidx_hbm.at[pl.ds(r0, nrows), pl.ds(0, BW)],
                               hbuf.at[pl.ds(0, nrows), :], sem_in)
        cp2 = pltpu.async_copy(widx_hbm.at[pl.ds(r0, nrows), pl.ds(0, BW)],
                               wbuf.at[pl.ds(0, nrows), :], sem_in)
        cp1.wait()
        cp2.wait()

        @pl.when(jnp.logical_not(first))
        def _drain():
            pltpu.make_async_copy(
                out_hbm.at[pl.ds(0, RPT), pl.ds(0, HPC), pl.ds(0, BW)],
                obuf, sem_out).wait()

        def gbody(v, _):
            off = v * 16
            cvs = []
            for r in range(nrows):
                c = hbuf[r, pl.ds(off, 16)] * NREL + wbuf[r, pl.ds(off, 16)]
                cvs.append(jnp.minimum(jnp.maximum(c, 0), TBL - 1))
            pgs = [plsc.load_gather(comb.at[pl.ds(hh * TBLP, TBL)], [cvs[0]])
                   for hh in range(HPC)]
            for r in range(1, nrows):
                ngs = []
                for hh in range(HPC):
                    g = plsc.load_gather(comb.at[pl.ds(hh * TBLP, TBL)],
                                         [cvs[r]])
                    obuf[r - 1, hh, pl.ds(off, 16)] = pgs[hh]
                    ngs.append(g)
                pgs = ngs
            for hh in range(HPC):
                obuf[nrows - 1, hh, pl.ds(off, 16)] = pgs[hh]
            return 0

        lax.fori_loop(0, VPR, gbody, 0)

        pltpu.async_copy(
            obuf.at[pl.ds(0, nrows), :, :],
            out_hbm.at[pl.ds(r0, nrows), pl.ds(cid * HPC, HPC), pl.ds(0, BW)],
            sem_out)

    def tile_body(i, _):
        do_tile((tid * 8 + i) * RPT, RPT, i == 0, HPC)
        return 0

    lax.fori_loop(0, 8, tile_body, 0)

    pltpu.make_async_copy(out_hbm.at[pl.ds(0, RPT), pl.ds(0, HPC), pl.ds(0, BW)],
                          obuf, sem_out).wait()

    @pl.when(tid == NS - 1)
    def _tail():
        do_tile(NRT * RPT, 1, True, 0)
        pltpu.make_async_copy(
            out_hbm.at[pl.ds(0, 1), pl.ds(0, HPC), pl.ds(0, BW)],
            obuf.at[pl.ds(0, 1), :, :], sem_out).wait()


def kernel(bias_high, bias_width, h_index, w_index):
    bh_t = jnp.zeros((NUM_HEADS, 80), jnp.float32).at[:, :NREL].set(
        bias_high.T).reshape(NUM_HEADS * 80)
    bw_t = jnp.zeros((NUM_HEADS, 80), jnp.float32).at[:, :NREL].set(
        bias_width.T).reshape(NUM_HEADS * 80)

    run = pl.kernel(
        _sc_body,
        out_type=jax.ShapeDtypeStruct((N, NUM_HEADS, N), jnp.float32),
        mesh=plsc.VectorSubcoreMesh(core_axis_name="c", subcore_axis_name="s",
                                    num_cores=NC, num_subcores=NS),
        compiler_params=pltpu.CompilerParams(use_tc_tiling_on_sc=True,
                                             needs_layout_passes=False),
        scratch_types=[
            pltpu.VMEM((NUM_HEADS * 80,), jnp.float32),
            pltpu.VMEM((NUM_HEADS * 80,), jnp.float32),
            pltpu.VMEM((COMB_WORDS,), jnp.float32),
            pltpu.VMEM((RPT, BW), jnp.int32),
            pltpu.VMEM((RPT, BW), jnp.int32),
            pltpu.VMEM((RPT, HPC, BW), jnp.float32),
            pltpu.SemaphoreType.DMA,
            pltpu.SemaphoreType.DMA,
        ],
    )
    out_ihj = run(bh_t, bw_t, h_index.astype(jnp.int32),
                  w_index.astype(jnp.int32))
    return jnp.transpose(out_ihj, (1, 0, 2))
